# baseline (speedup 1.0000x reference)
import functools

import jax
import jax.numpy as jnp
from jax import lax
from jax.experimental import pallas as pl
from jax.experimental.pallas import tpu as pltpu
from jax.experimental.pallas import tpu_sc as plsc


def _sc_worker_count():
    try:
        info = plsc.get_sparse_core_info()
        return info.num_cores, info.num_subcores
    except Exception:
        return 2, 16


def _sc_gather(user_table, users):
    batch, = users.shape
    _, dim = user_table.shape
    nc, ns = _sc_worker_count()
    nw = nc * ns
    b_per_w = batch // nw
    assert batch % nw == 0 and b_per_w % 8 == 0

    mesh = plsc.VectorSubcoreMesh(core_axis_name="c", subcore_axis_name="s")

    @functools.partial(
        pl.kernel,
        mesh=mesh,
        compiler_params=pltpu.CompilerParams(use_tc_tiling_on_sc=False),
        out_type=jax.ShapeDtypeStruct((batch, dim), jnp.float32),
        scratch_types=[
            pltpu.VMEM((b_per_w,), jnp.int32),
            pltpu.VMEM((b_per_w, dim), jnp.float32),
            pltpu.SemaphoreType.DMA,
        ],
    )
    def gather_kernel(table_hbm, idx_hbm, out_hbm, idx_v, rows_v, sem):
        wid = lax.axis_index("s") * nc + lax.axis_index("c")
        base = wid * b_per_w
        pltpu.sync_copy(idx_hbm.at[pl.ds(base, b_per_w)], idx_v)
        pltpu.async_copy(table_hbm.at[idx_v], rows_v, sem).wait()
        pltpu.sync_copy(rows_v, out_hbm.at[pl.ds(base, b_per_w)])

    return gather_kernel(user_table, users)


_ITEM_BLK = 1024
_RING = 4


def _mm_body(num_items, ue_ref, it_hbm, out_hbm,
             it_buf, buf, tail_in, tail_buf, in_sems, out_sems, tail_sems):
    ue = ue_ref[...]
    n_full = num_items // _ITEM_BLK
    tail = num_items % _ITEM_BLK

    def make_read(slot, off):
        return pltpu.make_async_copy(
            it_hbm.at[pl.ds(off, _ITEM_BLK), :],
            it_buf.at[slot],
            in_sems.at[slot],
        )

    def make_write(slot, off):
        return pltpu.make_async_copy(
            buf.at[slot],
            out_hbm.at[:, pl.ds(off, _ITEM_BLK)],
            out_sems.at[slot],
        )

    def score(it_block):
        return lax.dot_general(
            ue, it_block,
            (((1,), (1,)), ((), ())),
            preferred_element_type=jnp.float32,
        )

    for i in range(min(_RING, n_full)):
        make_read(i, i * _ITEM_BLK).start()

    for i in range(n_full):
        islot = i % _RING
        oslot = i % _RING
        make_read(islot, i * _ITEM_BLK).wait()
        if i >= _RING:
            make_write(oslot, (i - _RING) * _ITEM_BLK).wait()
        buf[oslot] = score(it_buf[islot])
        make_write(oslot, i * _ITEM_BLK).start()
        if i + _RING < n_full:
            make_read(islot, (i + _RING) * _ITEM_BLK).start()

    if tail:
        off = n_full * _ITEM_BLK
        tin = pltpu.make_async_copy(
            it_hbm.at[pl.ds(off, tail), :], tail_in, tail_sems.at[0])
        tin.start()
        tin.wait()
        tail_buf[...] = score(tail_in[...])
        tout = pltpu.make_async_copy(
            tail_buf, out_hbm.at[:, pl.ds(off, tail)], tail_sems.at[1])
        tout.start()
        tout.wait()

    for i in range(max(0, n_full - _RING), n_full):
        make_write(i % _RING, i * _ITEM_BLK).wait()


def _tc_scores(user_emb, item_table):
    batch, dim = user_emb.shape
    num_items, _ = item_table.shape
    tail = num_items % _ITEM_BLK or _ITEM_BLK
    return pl.pallas_call(
        functools.partial(_mm_body, num_items),
        in_specs=[
            pl.BlockSpec(memory_space=pltpu.VMEM),
            pl.BlockSpec(memory_space=pltpu.HBM),
        ],
        out_specs=pl.BlockSpec(memory_space=pltpu.HBM),
        out_shape=jax.ShapeDtypeStruct((batch, num_items), jnp.float32),
        scratch_shapes=[
            pltpu.VMEM((_RING, _ITEM_BLK, dim), jnp.float32),
            pltpu.VMEM((_RING, batch, _ITEM_BLK), jnp.float32),
            pltpu.VMEM((tail, dim), jnp.float32),
            pltpu.VMEM((batch, tail), jnp.float32),
            pltpu.SemaphoreType.DMA((_RING,)),
            pltpu.SemaphoreType.DMA((_RING,)),
            pltpu.SemaphoreType.DMA((2,)),
        ],
        compiler_params=pltpu.CompilerParams(
            vmem_limit_bytes=120 * 1024 * 1024,
        ),
    )(user_emb, item_table)


def kernel(users, user_table, item_table):
    user_emb = jnp.take(user_table, users, axis=0)
    return _tc_scores(user_emb, item_table)
